# baseline (speedup 1.0000x reference)
import functools

import jax
import jax.numpy as jnp
from jax import lax
from jax.experimental import pallas as pl
from jax.experimental.pallas import tpu as pltpu
from jax.experimental.pallas import tpu_sc as plsc

D_HALF = 256
N_TOK = 16384
N_SLICE = 8
NC, NS = 2, 16
NW = NC * NS
TOK_PER_W = N_TOK // (NW // N_SLICE)
C_TOK = 64
N_CHUNK = TOK_PER_W // C_TOK


def _sc_embed(x_i32, W1, W2, W3, W4):
    mesh = plsc.VectorSubcoreMesh(core_axis_name="c", subcore_axis_name="s")

    @functools.partial(
        pl.kernel,
        mesh=mesh,
        compiler_params=pltpu.CompilerParams(needs_layout_passes=False),
        out_type=jax.ShapeDtypeStruct((N_TOK, N_SLICE, D_HALF), jnp.float32),
        scratch_types=[
            pltpu.VMEM((256, D_HALF), jnp.float32),
            pltpu.VMEM((TOK_PER_W,), jnp.int32),
            [pltpu.SemaphoreType.DMA] * 2,
            pltpu.SemaphoreType.DMA,
        ],
    )
    def k(x_hbm, w1_hbm, w2_hbm, w3_hbm, w4_hbm, out_hbm,
          tab_v, x_v, ssems, tsem):
        wid = lax.axis_index("s") * NC + lax.axis_index("c")
        kh = wid % N_SLICE
        q = wid // N_SLICE
        tok0 = q * TOK_PER_W
        tab_k = kh // 2
        half = kh % 2
        col0 = half * D_HALF

        tab_copy = None
        for kk, w_hbm in enumerate((w1_hbm, w2_hbm, w3_hbm, w4_hbm)):
            @pl.when(tab_k == kk)
            def _():
                pltpu.async_copy(
                    w_hbm.at[:, pl.ds(col0, D_HALF)], tab_v, tsem
                )
        tab_copy = pltpu.make_async_copy(
            w1_hbm.at[:, pl.ds(col0, D_HALF)], tab_v, tsem
        )

        pltpu.sync_copy(x_hbm.at[pl.ds(tok0, TOK_PER_W)], x_v)

        shift = jnp.broadcast_to((tab_k * 8).astype(jnp.int32), (16,))
        tab_copy.wait()

        def send_chunk(c, buf):
            for g in range(C_TOK // 16):
                v = x_v[pl.ds(c * C_TOK + g * 16, 16)]
                b16 = lax.shift_right_logical(v, shift) & 255
                for t16 in range(16):
                    t = c * C_TOK + g * 16 + t16
                    pltpu.async_copy(
                        tab_v.at[b16[t16]],
                        out_hbm.at[tok0 + t, kh],
                        ssems[buf],
                    )

        def wait_chunk(buf):
            pltpu.make_async_copy(
                out_hbm.at[pl.ds(tok0, C_TOK), kh],
                out_hbm.at[pl.ds(tok0, C_TOK), kh],
                ssems[buf],
            ).wait()

        def loop_body(i, carry):
            c = 2 * i

            @pl.when(i > 0)
            def _():
                wait_chunk(0)

            send_chunk(c, 0)

            @pl.when(i > 0)
            def _():
                wait_chunk(1)

            send_chunk(c + 1, 1)
            return carry

        lax.fori_loop(0, N_CHUNK // 2, loop_body, 0)
        wait_chunk(0)
        wait_chunk(1)

    return k(x_i32, W1, W2, W3, W4)


@jax.jit
def kernel(x, W1, W2, W3, W4):
    x_i32 = lax.bitcast_convert_type(x.reshape(-1), jnp.int32)
    out = _sc_embed(x_i32, W1, W2, W3, W4)
    return out.reshape(x.shape[0], x.shape[1], N_SLICE * D_HALF)
